# Initial kernel scaffold; baseline (speedup 1.0000x reference)
import jax
import jax.numpy as jnp
from jax.experimental import pallas as pl


def kernel(Z, R, atom_mask, emb, W1, W2, W3):
    raise NotImplementedError("write your pallas kernel here")


# speedup vs baseline: 1.0650x; 1.0650x over previous
import jax
import jax.numpy as jnp
import numpy as np
from jax.experimental import pallas as pl

B, N, D, NB = 128, 64, 128, 16
SIGMA = 0.25
BB = 8

_CENTERS = np.linspace(0.0, 4.0, NB).astype(np.float32)
_INV2S2 = float(1.0 / (2.0 * SIGMA * SIGMA))


def _conv_block_kernel(z_ref, r_ref, m_ref, emb_ref, w1_ref, w2_ref, w3_ref,
                       out_ref):
    f32 = jnp.float32
    hi = jax.lax.Precision.HIGHEST

    zf = z_ref[...]
    cols = jax.lax.broadcasted_iota(jnp.int32, (BB * N, 16), 1)
    oh = (cols == zf).astype(f32)
    f0 = jnp.dot(oh, emb_ref[...], precision=hi)

    mf = m_ref[...]
    m3 = mf.reshape(BB, N, 1)
    inv3 = jax.lax.rsqrt(jnp.sum(m3, axis=1, keepdims=True))

    rp = r_ref[...]
    ones = jnp.ones((N, 8), f32)
    basis = []
    for b in range(BB):
        rb = rp[b]
        rsq = rb * rb
        sq = jnp.sum(rsq, axis=1, keepdims=True)
        g = jax.lax.dot_general(rb, rb, (((1,), (1,)), ((), ())), precision=hi)
        sqj = jax.lax.dot_general(ones, rsq, (((1,), (1,)), ((), ())),
                                  precision=hi)
        d2 = jnp.maximum(sq + sqj - 2.0 * g, 0.0)
        dist = jnp.sqrt(d2 + 1e-12)
        basis.append([jnp.exp(-((dist - c) ** 2) * _INV2S2) for c in _CENTERS])

    def conv(feat_flat, w_ref):
        fs = (feat_flat.reshape(BB, N, D) * inv3).reshape(BB * N, D)
        msg = jnp.dot(fs, w_ref[...])
        outs = []
        for b in range(BB):
            mb = msg[b * N:(b + 1) * N]
            acc = None
            for n in range(NB):
                t = jnp.dot(basis[b][n], mb[:, n * D:(n + 1) * D])
                acc = t if acc is None else acc + t
            outs.append(acc)
        return jnp.concatenate(outs, axis=0)

    feat = jnp.maximum(conv(f0, w1_ref), 0.0)
    for w_ref in (w2_ref, w3_ref):
        new = jnp.maximum(conv(feat, w_ref), 0.0) * mf
        feat = feat + new
    out_ref[...] = feat.reshape(BB, N, D)


def kernel(Z, R, atom_mask, emb, W1, W2, W3):
    Zf = Z.astype(jnp.int32).reshape(B * N, 1)
    Mf = atom_mask.astype(jnp.float32).reshape(B * N, 1)
    Rp = jnp.pad(R.astype(jnp.float32), ((0, 0), (0, 0), (0, 5)))
    emb16 = jnp.pad(emb, ((0, 16 - emb.shape[0]), (0, 0)))

    def wr(W):
        return jnp.transpose(W, (1, 0, 2)).reshape(D, NB * D)

    return pl.pallas_call(
        _conv_block_kernel,
        grid=(B // BB,),
        in_specs=[
            pl.BlockSpec((BB * N, 1), lambda i: (i, 0)),
            pl.BlockSpec((BB, N, 8), lambda i: (i, 0, 0)),
            pl.BlockSpec((BB * N, 1), lambda i: (i, 0)),
            pl.BlockSpec((16, D), lambda i: (0, 0)),
            pl.BlockSpec((D, NB * D), lambda i: (0, 0)),
            pl.BlockSpec((D, NB * D), lambda i: (0, 0)),
            pl.BlockSpec((D, NB * D), lambda i: (0, 0)),
        ],
        out_specs=pl.BlockSpec((BB, N, D), lambda i: (i, 0, 0)),
        out_shape=jax.ShapeDtypeStruct((B, N, D), jnp.float32),
    )(Zf, Rp, Mf, emb16, wr(W1), wr(W2), wr(W3))
